# baseline (device time: 1071150 ns/iter reference)
import jax
import jax.numpy as jnp
from jax import lax
from jax.experimental import pallas as pl
from jax.experimental.pallas import tpu as pltpu

M_SHARD = 8192
N_COLS = 1024
Y_SIZE = 2
N_CHUNKS = 8
ROWS_PER_CHUNK = M_SHARD // N_CHUNKS


def kernel(x):
    m, n = x.shape
    assert (m, n) == (M_SHARD, N_COLS), (m, n)

    def body(x_ref, out_ref, copy_sem, send_sems, recv_sems):
        my_x = lax.axis_index("x")
        my_y = lax.axis_index("y")
        my_z = lax.axis_index("z")

        local = pltpu.make_async_copy(
            x_ref,
            out_ref.at[pl.ds(my_y * M_SHARD, M_SHARD), :],
            copy_sem,
        )
        local.start()

        rdmas = []
        for c in range(N_CHUNKS):
            rows = pl.ds(my_y * M_SHARD + c * ROWS_PER_CHUNK, ROWS_PER_CHUNK)
            rdma = pltpu.make_async_remote_copy(
                src_ref=x_ref.at[pl.ds(c * ROWS_PER_CHUNK, ROWS_PER_CHUNK), :],
                dst_ref=out_ref.at[rows, :],
                send_sem=send_sems.at[c],
                recv_sem=recv_sems.at[c],
                device_id=(my_x, 1 - my_y, my_z),
                device_id_type=pl.DeviceIdType.MESH,
            )
            rdma.start()
            rdmas.append(rdma)
        for rdma in rdmas:
            rdma.wait()
        local.wait()

    return pl.pallas_call(
        body,
        out_shape=jax.ShapeDtypeStruct((Y_SIZE * M_SHARD, N_COLS), x.dtype),
        in_specs=[pl.BlockSpec(memory_space=pltpu.MemorySpace.HBM)],
        out_specs=pl.BlockSpec(memory_space=pltpu.MemorySpace.HBM),
        scratch_shapes=[
            pltpu.SemaphoreType.DMA,
            pltpu.SemaphoreType.DMA((N_CHUNKS,)),
            pltpu.SemaphoreType.DMA((N_CHUNKS,)),
        ],
    )(x)
